# baseline (speedup 1.0000x reference)
import jax
import jax.numpy as jnp
from jax import lax
from jax.experimental import pallas as pl
from jax.experimental.pallas import tpu as pltpu
from jax.experimental.pallas import tpu_sc as plsc

N_NODES = 10000
N_EDGES = 320000
D = 128

NC = 2
NS = 16
NW = NC * NS
C = 112

E_PER_W = N_EDGES // NW
N_FULL = E_PER_W // C
TAIL = E_PER_W - N_FULL * C
K = 15
N_PH_FULL = 5
K_LAST = N_FULL - N_PH_FULL * K
NBUF = 3
ROWS_A = 632
ROWS_B = N_NODES - 15 * ROWS_A
IDX_WORDS = K * C


def _elu_body(x_ref, w_ref, o_ref):
    x = x_ref[...] * w_ref[...]
    o_ref[...] = 2.0 * jnp.where(x > 0, x, jnp.exp(jnp.minimum(x, 0.0)) - 1.0)


def _elu_tc(graph_embedding, weight):
    return pl.pallas_call(
        _elu_body,
        out_shape=jax.ShapeDtypeStruct((N_NODES, D), jnp.float32),
    )(graph_embedding, weight)


def _combine_body(p_ref, o_ref):
    o_ref[...] = p_ref[0] + p_ref[1]


def _combine_tc(partials):
    return pl.pallas_call(
        _combine_body,
        out_shape=jax.ShapeDtypeStruct((N_NODES, D), jnp.float32),
    )(partials)


def _sc_body(emb_hbm, idx_hbm, out_hbm,
             src_v, dst1d_v, dst_v, dstt_v, rows, acc, gsem, ssem):
    cid = lax.axis_index("c")
    sid = lax.axis_index("s")
    base = (cid * NS + sid) * E_PER_W

    zero16 = jnp.zeros((16,), jnp.float32)

    @pl.loop(0, C)
    def _zero_rows(r):
        for c16 in range(D // 16):
            rows[0][r, pl.ds(c16 * 16, 16)] = zero16

    row0 = sid * ROWS_A

    def zero_span(n_rows):
        for k in range(n_rows // C):
            pltpu.sync_copy(rows[0], acc.at[pl.ds(row0 + k * C, C)])
        rem = n_rows % C
        pltpu.sync_copy(rows[0].at[pl.ds(0, rem)],
                        acc.at[pl.ds(row0 + (n_rows // C) * C, rem)])

    @pl.when(sid < NS - 1)
    def _():
        zero_span(ROWS_A)

    @pl.when(sid == NS - 1)
    def _():
        zero_span(ROWS_B)

    plsc.subcore_barrier()

    def start_gather(j, buf):
        pltpu.async_copy(emb_hbm.at[src_v.at[pl.ds(j * C, C)]], buf, gsem)

    def wait_gather(j, buf):
        pltpu.make_async_copy(
            emb_hbm.at[src_v.at[pl.ds(j * C, C)]], buf, gsem).wait()

    def start_scatter(j, buf):
        pltpu.async_copy(buf, acc.at[dst_v.at[j]], ssem, add=True)

    def wait_scatter(j, buf):
        pltpu.make_async_copy(buf, acc.at[dst_v.at[j]], ssem).wait()

    def repack_dst(j):
        for k in range(C // 16):
            dst_v[j, pl.ds(k * 16, 16)] = dst1d_v[pl.ds(j * C + k * 16, 16)]

    def chunk_body(j, b, n):
        wait_gather(j, rows[b])

        @pl.when(j >= 2)
        def _():
            wait_scatter(j - 2, rows[(b + 1) % NBUF])

        @pl.when(j + 1 < n)
        def _():
            start_gather(j + 1, rows[(b + 1) % NBUF])

        repack_dst(j)
        start_scatter(j, rows[b])

    def run_phase(p, n):
        words = n * C
        pltpu.sync_copy(idx_hbm.at[pl.ds(base + p * IDX_WORDS, words)],
                        src_v.at[pl.ds(0, words)])
        pltpu.sync_copy(idx_hbm.at[pl.ds(N_EDGES + base + p * IDX_WORDS,
                                         words)],
                        dst1d_v.at[pl.ds(0, words)])
        start_gather(0, rows[0])

        @pl.loop(0, n // NBUF)
        def _trips(t):
            for b in range(NBUF):
                chunk_body(NBUF * t + b, b, n)

        for j in range((n // NBUF) * NBUF, n):
            chunk_body(j, j % NBUF, n)

        wait_scatter(n - 2, rows[(n - 2) % NBUF])
        wait_scatter(n - 1, rows[(n - 1) % NBUF])

    for p in range(N_PH_FULL):
        run_phase(p, K)
    run_phase(N_PH_FULL, K_LAST)

    toff = N_PH_FULL * IDX_WORDS + K_LAST * C
    pltpu.sync_copy(idx_hbm.at[pl.ds(base + toff, TAIL)],
                    src_v.at[pl.ds(0, TAIL)])
    pltpu.sync_copy(idx_hbm.at[pl.ds(N_EDGES + base + toff, TAIL)],
                    dst1d_v.at[pl.ds(0, TAIL)])
    for k in range(TAIL // 16):
        dstt_v[0, pl.ds(k * 16, 16)] = dst1d_v[pl.ds(k * 16, 16)]
    pltpu.async_copy(emb_hbm.at[src_v.at[pl.ds(0, TAIL)]],
                     rows[0].at[pl.ds(0, TAIL)], gsem)
    pltpu.make_async_copy(emb_hbm.at[src_v.at[pl.ds(0, TAIL)]],
                          rows[0].at[pl.ds(0, TAIL)], gsem).wait()
    pltpu.sync_copy(rows[0].at[pl.ds(0, TAIL)],
                    acc.at[dstt_v.at[0]], add=True)

    plsc.subcore_barrier()

    @pl.when(sid < NS - 1)
    def _():
        pltpu.sync_copy(acc.at[pl.ds(row0, ROWS_A)],
                        out_hbm.at[cid].at[pl.ds(row0, ROWS_A)])

    @pl.when(sid == NS - 1)
    def _():
        pltpu.sync_copy(acc.at[pl.ds(row0, ROWS_B)],
                        out_hbm.at[cid].at[pl.ds(row0, ROWS_B)])


def _sc_scatter(emb2, idx_flat):
    mesh = plsc.VectorSubcoreMesh(core_axis_name="c", subcore_axis_name="s",
                                  num_cores=NC, num_subcores=NS)
    return pl.kernel(
        _sc_body,
        out_type=jax.ShapeDtypeStruct((NC, N_NODES, D), jnp.float32),
        mesh=mesh,
        scratch_types=[
            pltpu.VMEM((IDX_WORDS,), jnp.int32),
            pltpu.VMEM((IDX_WORDS,), jnp.int32),
            pltpu.VMEM((K, C), jnp.int32),
            pltpu.VMEM((8, TAIL), jnp.int32),
            [pltpu.VMEM((C, D), jnp.float32)] * NBUF,
            pltpu.VMEM_SHARED((N_NODES, D), jnp.float32),
            pltpu.SemaphoreType.DMA,
            pltpu.SemaphoreType.DMA,
        ],
    )(emb2, idx_flat)


def kernel(graph_embedding, edge_index, e_feat, weight):
    del e_feat
    idx_flat = edge_index.astype(jnp.int32).reshape(-1)
    emb2 = _elu_tc(graph_embedding, weight)
    partials = _sc_scatter(emb2, idx_flat)
    return _combine_tc(partials)


# speedup vs baseline: 1.3275x; 1.3275x over previous
import jax
import jax.numpy as jnp
from jax import lax
from jax.experimental import pallas as pl
from jax.experimental.pallas import tpu as pltpu
from jax.experimental.pallas import tpu_sc as plsc

N_NODES = 10000
N_EDGES = 320000
D = 128

NC = 2
NS = 16
NW = NC * NS
C = 112

E_PER_W = N_EDGES // NW
N_FULL = E_PER_W // C
TAIL = E_PER_W - N_FULL * C
K = 14
N_PH_FULL = 6
K_LAST = N_FULL - N_PH_FULL * K
NBUF = 3
ROWS_A = 632
ROWS_B = N_NODES - 15 * ROWS_A
IDX_WORDS = K * C


def _elu_body(x_ref, w_ref, o_ref):
    x = x_ref[...] * w_ref[...]
    o_ref[...] = 2.0 * jnp.where(x > 0, x, jnp.exp(jnp.minimum(x, 0.0)) - 1.0)


def _elu_tc(graph_embedding, weight):
    return pl.pallas_call(
        _elu_body,
        out_shape=jax.ShapeDtypeStruct((N_NODES, D), jnp.float32),
    )(graph_embedding, weight)


def _combine_body(p_ref, o_ref):
    o_ref[...] = p_ref[0] + p_ref[1]


def _combine_tc(partials):
    return pl.pallas_call(
        _combine_body,
        out_shape=jax.ShapeDtypeStruct((N_NODES, D), jnp.float32),
    )(partials)


def _sc_body(emb_hbm, idx_hbm, out_hbm,
             src_v, src_v2, dst1d_v, dst_v, dstt_v, rows, acc,
             gsem, ssem, isem):
    cid = lax.axis_index("c")
    sid = lax.axis_index("s")
    base = (cid * NS + sid) * E_PER_W

    zero16 = jnp.zeros((16,), jnp.float32)

    @pl.loop(0, C)
    def _zero_rows(r):
        for c16 in range(D // 16):
            rows[0][r, pl.ds(c16 * 16, 16)] = zero16

    row0 = sid * ROWS_A

    def zero_span(n_rows):
        for k in range(n_rows // C):
            pltpu.sync_copy(rows[0], acc.at[pl.ds(row0 + k * C, C)])
        rem = n_rows % C
        pltpu.sync_copy(rows[0].at[pl.ds(0, rem)],
                        acc.at[pl.ds(row0 + (n_rows // C) * C, rem)])

    @pl.when(sid < NS - 1)
    def _():
        zero_span(ROWS_A)

    @pl.when(sid == NS - 1)
    def _():
        zero_span(ROWS_B)

    plsc.subcore_barrier()

    def start_gather(j, buf, src_ref):
        pltpu.async_copy(emb_hbm.at[src_ref.at[pl.ds(j * C, C)]], buf, gsem)

    def wait_gather(j, buf, src_ref):
        pltpu.make_async_copy(
            emb_hbm.at[src_ref.at[pl.ds(j * C, C)]], buf, gsem).wait()

    def start_scatter(j, buf):
        pltpu.async_copy(buf, acc.at[dst_v.at[j]], ssem, add=True)

    def wait_scatter(j, buf):
        pltpu.make_async_copy(buf, acc.at[dst_v.at[j]], ssem).wait()

    def repack_dst(j):
        for k in range(C // 16):
            dst_v[j, pl.ds(k * 16, 16)] = dst1d_v[pl.ds(j * C + k * 16, 16)]

    def chunk_body(j, b, n, src_ref):
        wait_gather(j, rows[b], src_ref)

        @pl.when(j >= 1)
        def _():
            wait_scatter(j - 1, rows[(b + 2) % NBUF])

        @pl.when(j + 2 < n)
        def _():
            start_gather(j + 2, rows[(b + 2) % NBUF], src_ref)

        repack_dst(j)
        start_scatter(j, rows[b])

    srcs = [src_v, src_v2]
    phase_lens = [K] * N_PH_FULL + [K_LAST]

    def stage_src(p, ref, sync):
        copy = (pltpu.sync_copy if sync else
                lambda s, d: pltpu.async_copy(s, d, isem))
        words = phase_lens[p] * C
        copy(idx_hbm.at[pl.ds(base + p * IDX_WORDS, words)],
             ref.at[pl.ds(0, words)])

    def wait_src(p, ref):
        words = phase_lens[p] * C
        pltpu.make_async_copy(
            idx_hbm.at[pl.ds(base + p * IDX_WORDS, words)],
            ref.at[pl.ds(0, words)], isem).wait()

    stage_src(0, srcs[0], sync=True)
    start_gather(0, rows[0], srcs[0])
    start_gather(1, rows[1], srcs[0])

    for p, n in enumerate(phase_lens):
        src_ref = srcs[p % 2]
        if p + 1 < len(phase_lens):
            stage_src(p + 1, srcs[(p + 1) % 2], sync=False)
        pltpu.sync_copy(idx_hbm.at[pl.ds(N_EDGES + base + p * IDX_WORDS,
                                         n * C)],
                        dst1d_v.at[pl.ds(0, n * C)])

        @pl.loop(0, n // NBUF)
        def _trips(t):
            for b in range(NBUF):
                chunk_body(NBUF * t + b, b, n, src_ref)

        for j in range((n // NBUF) * NBUF, n):
            chunk_body(j, j % NBUF, n, src_ref)

        wait_scatter(n - 1, rows[(n - 1) % NBUF])
        if p + 1 < len(phase_lens):
            nref = srcs[(p + 1) % 2]
            wait_src(p + 1, nref)
            start_gather(0, rows[0], nref)
            start_gather(1, rows[1], nref)

    toff = N_FULL * C
    pltpu.sync_copy(idx_hbm.at[pl.ds(base + toff, TAIL)],
                    src_v.at[pl.ds(0, TAIL)])
    pltpu.sync_copy(idx_hbm.at[pl.ds(N_EDGES + base + toff, TAIL)],
                    dst1d_v.at[pl.ds(0, TAIL)])
    for k in range(TAIL // 16):
        dstt_v[0, pl.ds(k * 16, 16)] = dst1d_v[pl.ds(k * 16, 16)]
    pltpu.async_copy(emb_hbm.at[src_v.at[pl.ds(0, TAIL)]],
                     rows[0].at[pl.ds(0, TAIL)], gsem)
    pltpu.make_async_copy(emb_hbm.at[src_v.at[pl.ds(0, TAIL)]],
                          rows[0].at[pl.ds(0, TAIL)], gsem).wait()
    pltpu.sync_copy(rows[0].at[pl.ds(0, TAIL)],
                    acc.at[dstt_v.at[0]], add=True)

    plsc.subcore_barrier()

    @pl.when(sid < NS - 1)
    def _():
        pltpu.sync_copy(acc.at[pl.ds(row0, ROWS_A)],
                        out_hbm.at[cid].at[pl.ds(row0, ROWS_A)])

    @pl.when(sid == NS - 1)
    def _():
        pltpu.sync_copy(acc.at[pl.ds(row0, ROWS_B)],
                        out_hbm.at[cid].at[pl.ds(row0, ROWS_B)])


def _sc_scatter(emb2, idx_flat):
    mesh = plsc.VectorSubcoreMesh(core_axis_name="c", subcore_axis_name="s",
                                  num_cores=NC, num_subcores=NS)
    return pl.kernel(
        _sc_body,
        out_type=jax.ShapeDtypeStruct((NC, N_NODES, D), jnp.float32),
        mesh=mesh,
        scratch_types=[
            pltpu.VMEM((IDX_WORDS,), jnp.int32),
            pltpu.VMEM((IDX_WORDS,), jnp.int32),
            pltpu.VMEM((IDX_WORDS,), jnp.int32),
            pltpu.VMEM((K, C), jnp.int32),
            pltpu.VMEM((8, TAIL), jnp.int32),
            [pltpu.VMEM((C, D), jnp.float32)] * NBUF,
            pltpu.VMEM_SHARED((N_NODES, D), jnp.float32),
            pltpu.SemaphoreType.DMA,
            pltpu.SemaphoreType.DMA,
            pltpu.SemaphoreType.DMA,
        ],
    )(emb2, idx_flat)


def kernel(graph_embedding, edge_index, e_feat, weight):
    del e_feat
    idx_flat = edge_index.astype(jnp.int32).reshape(-1)
    emb2 = _elu_tc(graph_embedding, weight)
    partials = _sc_scatter(emb2, idx_flat)
    return _combine_tc(partials)
